# baseline (speedup 1.0000x reference)
import functools

import jax
import jax.numpy as jnp
from jax import lax
from jax.experimental import pallas as pl
from jax.experimental.pallas import tpu as pltpu
from jax.experimental.pallas import tpu_sc as plsc

N = 10000
E = 320000
NFEAT = 128
NHID = 128
NCLASS = 40

NPAD = 10240
DUMMY = N
CH = 128
NC = 2
NS = 16
NW = NC * NS
KCH = 80
K0 = 152
K1 = 2 * KCH - K0
EPAD = KCH * CH * NW
RPT = NPAD // NS
BLK = 1024
IDXG = 8


def _degree_kernel():
    mesh = plsc.VectorSubcoreMesh(core_axis_name="c", subcore_axis_name="s",
                                  num_cores=NC, num_subcores=NS)

    @functools.partial(
        pl.kernel, mesh=mesh,
        out_type=jax.ShapeDtypeStruct((NC * NPAD, NHID), jnp.float32),
        scratch_types=[
            pltpu.VMEM((KCH, CH), jnp.int32),
            pltpu.VMEM((CH, NHID), jnp.float32),
            pltpu.VMEM_SHARED((NPAD, NHID), jnp.float32),
        ],
    )
    def deg_kernel(rowc_hbm, zeros_hbm, ones_hbm, out_hbm, row_v, ones_v, acc_sh):
        c = lax.axis_index("c")
        s = lax.axis_index("s")
        wid = s * NC + c
        pltpu.sync_copy(zeros_hbm.at[pl.ds(s * RPT, RPT)],
                        acc_sh.at[pl.ds(s * RPT, RPT)])
        pltpu.sync_copy(rowc_hbm.at[pl.ds(wid * KCH, KCH)], row_v)
        pltpu.sync_copy(ones_hbm, ones_v)
        plsc.subcore_barrier()

        def body(j, carry):
            pltpu.sync_copy(ones_v, acc_sh.at[row_v.at[j]], add=True)
            return carry

        lax.fori_loop(0, KCH, body, 0)
        plsc.subcore_barrier()
        pltpu.sync_copy(acc_sh.at[pl.ds(s * RPT, RPT)],
                        out_hbm.at[pl.ds(c * NPAD + s * RPT, RPT)])

    return deg_kernel


def _scatter_kernel(D):
    mesh = plsc.VectorSubcoreMesh(core_axis_name="c", subcore_axis_name="s",
                                  num_cores=NC, num_subcores=NS)

    @functools.partial(
        pl.kernel, mesh=mesh,
        out_type=jax.ShapeDtypeStruct((NC * NPAD, D), jnp.float32),
        scratch_types=[
            pltpu.VMEM((IDXG, CH), jnp.int32),
            pltpu.VMEM((IDXG, CH), jnp.int32),
            pltpu.VMEM((CH, D), jnp.float32),
            pltpu.VMEM((CH, D), jnp.float32),
            pltpu.VMEM_SHARED((NPAD, D), jnp.float32),
            pltpu.SemaphoreType.DMA,
            pltpu.SemaphoreType.DMA,
        ],
    )
    def scatter_kernel(hs_hbm, colc_hbm, rowc_hbm, zeros_hbm, out_hbm,
                       col_v, row_v, rows0, rows1, acc_sh, sem0, sem1):
        rows_v = (rows0, rows1)
        sems = (sem0, sem1)
        c = lax.axis_index("c")
        s = lax.axis_index("s")
        pltpu.sync_copy(zeros_hbm.at[pl.ds(s * RPT, RPT)],
                        acc_sh.at[pl.ds(s * RPT, RPT)])
        plsc.subcore_barrier()

        def body_at(tile_base):
            def body(g, carry):
                base = tile_base + g * IDXG
                pltpu.sync_copy(colc_hbm.at[pl.ds(base, IDXG)], col_v)
                pltpu.sync_copy(rowc_hbm.at[pl.ds(base, IDXG)], row_v)
                pltpu.async_copy(hs_hbm.at[col_v.at[0]], rows_v[0], sems[0])
                for t in range(IDXG):
                    b = t % 2
                    pltpu.make_async_copy(
                        hs_hbm.at[col_v.at[t]], rows_v[b], sems[b]).wait()
                    if t + 1 < IDXG:
                        pltpu.async_copy(
                            hs_hbm.at[col_v.at[t + 1]], rows_v[1 - b],
                            sems[1 - b])
                    pltpu.sync_copy(rows_v[b], acc_sh.at[row_v.at[t]],
                                    add=True)
                return carry
            return body

        @pl.when(c == 0)
        def _():
            lax.fori_loop(0, K0 // IDXG, body_at(s * K0), 0)

        @pl.when(c == 1)
        def _():
            lax.fori_loop(0, K1 // IDXG, body_at(NS * K0 + s * K1), 0)

        plsc.subcore_barrier()
        pltpu.sync_copy(acc_sh.at[pl.ds(s * RPT, RPT)],
                        out_hbm.at[pl.ds(c * NPAD + s * RPT, RPT)])

    return scatter_kernel


def _dinv_block(degp_ref):
    d = degp_ref[0, :, :1] + degp_ref[1, :, :1] + 1.0
    return lax.rsqrt(d)


def _first_body(x_ref, w_ref, degp_ref, o_ref):
    dinv = _dinv_block(degp_ref)
    o_ref[...] = jnp.dot(x_ref[...], w_ref[...],
                         preferred_element_type=jnp.float32) * dinv


def _mid_body(p_ref, hs_ref, b_ref, w_ref, degp_ref, o_ref):
    dinv = _dinv_block(degp_ref)
    h = dinv * (p_ref[0] + p_ref[1] + hs_ref[...]) + b_ref[...]
    h = jnp.maximum(h, 0.0)
    o_ref[...] = jnp.dot(h, w_ref[...],
                         preferred_element_type=jnp.float32) * dinv


def _final_body(p_ref, hs_ref, b_ref, degp_ref, o_ref):
    dinv = _dinv_block(degp_ref)
    o_ref[...] = dinv * (p_ref[0] + p_ref[1] + hs_ref[...]) + b_ref[...]


def _row_spec(d):
    return pl.BlockSpec((BLK, d), lambda i: (i, 0))


def _part_spec(d):
    return pl.BlockSpec((2, BLK, d), lambda i: (0, i, 0))


def _full_spec(d0, d1):
    return pl.BlockSpec((d0, d1), lambda i: (0, 0))


def _tc_first(x, w, degp):
    return pl.pallas_call(
        _first_body,
        grid=(NPAD // BLK,),
        in_specs=[_row_spec(NFEAT), _full_spec(NFEAT, NHID), _part_spec(NHID)],
        out_specs=_row_spec(NHID),
        out_shape=jax.ShapeDtypeStruct((NPAD, NHID), jnp.float32),
    )(x, w, degp)


def _tc_mid(p, hs, b, w, degp, dout):
    din = hs.shape[1]
    return pl.pallas_call(
        _mid_body,
        grid=(NPAD // BLK,),
        in_specs=[_part_spec(din), _row_spec(din), _full_spec(1, din),
                  _full_spec(din, dout), _part_spec(NHID)],
        out_specs=_row_spec(dout),
        out_shape=jax.ShapeDtypeStruct((NPAD, dout), jnp.float32),
    )(p, hs, b, w, degp)


def _tc_final(p, hs, b, degp):
    d = hs.shape[1]
    return pl.pallas_call(
        _final_body,
        grid=(NPAD // BLK,),
        in_specs=[_part_spec(d), _row_spec(d), _full_spec(1, d),
                  _part_spec(NHID)],
        out_specs=_row_spec(d),
        out_shape=jax.ShapeDtypeStruct((NPAD, d), jnp.float32),
    )(p, hs, b, degp)


def kernel(x, edge_index, W0, b0, W1, b1, W2, b2):
    row = edge_index[0]
    col = edge_index[1]
    pad = EPAD - E
    rowc = jnp.concatenate(
        [row, jnp.full((pad,), DUMMY, jnp.int32)]).reshape(NW * KCH, CH)
    colc = jnp.concatenate(
        [col, jnp.zeros((pad,), jnp.int32)]).reshape(NW * KCH, CH)

    zeros128 = jnp.zeros((NPAD, NHID), jnp.float32)
    ones128 = jnp.ones((CH, NHID), jnp.float32)

    degp = _degree_kernel()(rowc, zeros128, ones128).reshape(2, NPAD, NHID)

    xp = jnp.pad(x, ((0, NPAD - N), (0, 0)))
    W2p = jnp.pad(W2, ((0, 0), (0, 128 - NCLASS)))
    b2p = jnp.pad(b2, (0, 128 - NCLASS))

    scat128 = _scatter_kernel(NHID)

    hs0 = _tc_first(xp, W0, degp)
    p1 = scat128(hs0, colc, rowc, zeros128).reshape(2, NPAD, NHID)
    hs1 = _tc_mid(p1, hs0, b0.reshape(1, -1), W1, degp, NHID)
    p2 = scat128(hs1, colc, rowc, zeros128).reshape(2, NPAD, NHID)
    hs2 = _tc_mid(p2, hs1, b1.reshape(1, -1), W2p, degp, 128)
    p3 = scat128(hs2, colc, rowc, zeros128).reshape(2, NPAD, 128)
    out = _tc_final(p3, hs2, b2p.reshape(1, -1), degp)
    return out[:N, :NCLASS]


# speedup vs baseline: 1.0425x; 1.0425x over previous
import functools

import jax
import jax.numpy as jnp
from jax import lax
from jax.experimental import pallas as pl
from jax.experimental.pallas import tpu as pltpu
from jax.experimental.pallas import tpu_sc as plsc

N = 10000
E = 320000
NFEAT = 128
NHID = 128
NCLASS = 40

NPAD = 10240
DUMMY = N
CH = 128
NC = 2
NS = 16
NW = NC * NS
KCH = 80
K0 = 144
K1 = 2 * KCH - K0
EPAD = KCH * CH * NW
RPT = NPAD // NS
BLK = 1024
IDXG = 8


def _degree_kernel():
    mesh = plsc.VectorSubcoreMesh(core_axis_name="c", subcore_axis_name="s",
                                  num_cores=NC, num_subcores=NS)

    @functools.partial(
        pl.kernel, mesh=mesh,
        out_type=jax.ShapeDtypeStruct((NC * NPAD, NHID), jnp.float32),
        scratch_types=[
            pltpu.VMEM((KCH, CH), jnp.int32),
            pltpu.VMEM((CH, NHID), jnp.float32),
            pltpu.VMEM_SHARED((NPAD, NHID), jnp.float32),
        ],
    )
    def deg_kernel(rowc_hbm, zeros_hbm, ones_hbm, out_hbm, row_v, ones_v, acc_sh):
        c = lax.axis_index("c")
        s = lax.axis_index("s")
        wid = s * NC + c
        pltpu.sync_copy(zeros_hbm.at[pl.ds(s * RPT, RPT)],
                        acc_sh.at[pl.ds(s * RPT, RPT)])
        pltpu.sync_copy(rowc_hbm.at[pl.ds(wid * KCH, KCH)], row_v)
        pltpu.sync_copy(ones_hbm, ones_v)
        plsc.subcore_barrier()

        def body(j, carry):
            pltpu.sync_copy(ones_v, acc_sh.at[row_v.at[j]], add=True)
            return carry

        lax.fori_loop(0, KCH, body, 0)
        plsc.subcore_barrier()
        pltpu.sync_copy(acc_sh.at[pl.ds(s * RPT, RPT)],
                        out_hbm.at[pl.ds(c * NPAD + s * RPT, RPT)])

    return deg_kernel


def _scatter_kernel(D):
    mesh = plsc.VectorSubcoreMesh(core_axis_name="c", subcore_axis_name="s",
                                  num_cores=NC, num_subcores=NS)

    @functools.partial(
        pl.kernel, mesh=mesh,
        out_type=jax.ShapeDtypeStruct((NC * NPAD, D), jnp.float32),
        scratch_types=[
            pltpu.VMEM((IDXG, CH), jnp.int32),
            pltpu.VMEM((IDXG, CH), jnp.int32),
            pltpu.VMEM((CH, D), jnp.float32),
            pltpu.VMEM((CH, D), jnp.float32),
            pltpu.VMEM_SHARED((NPAD, D), jnp.float32),
            pltpu.SemaphoreType.DMA,
            pltpu.SemaphoreType.DMA,
        ],
    )
    def scatter_kernel(hs_hbm, colc_hbm, rowc_hbm, zeros_hbm, out_hbm,
                       col_v, row_v, rows0, rows1, acc_sh, sem0, sem1):
        rows_v = (rows0, rows1)
        sems = (sem0, sem1)
        c = lax.axis_index("c")
        s = lax.axis_index("s")
        pltpu.sync_copy(zeros_hbm.at[pl.ds(s * RPT, RPT)],
                        acc_sh.at[pl.ds(s * RPT, RPT)])
        plsc.subcore_barrier()

        def body_at(tile_base):
            def body(g, carry):
                base = tile_base + g * IDXG
                pltpu.sync_copy(colc_hbm.at[pl.ds(base, IDXG)], col_v)
                pltpu.sync_copy(rowc_hbm.at[pl.ds(base, IDXG)], row_v)
                pltpu.async_copy(hs_hbm.at[col_v.at[0]], rows_v[0], sems[0])
                for t in range(IDXG):
                    b = t % 2
                    pltpu.make_async_copy(
                        hs_hbm.at[col_v.at[t]], rows_v[b], sems[b]).wait()
                    if t + 1 < IDXG:
                        pltpu.async_copy(
                            hs_hbm.at[col_v.at[t + 1]], rows_v[1 - b],
                            sems[1 - b])
                    pltpu.sync_copy(rows_v[b], acc_sh.at[row_v.at[t]],
                                    add=True)
                return carry
            return body

        @pl.when(c == 0)
        def _():
            lax.fori_loop(0, K0 // IDXG, body_at(s * K0), 0)

        @pl.when(c == 1)
        def _():
            lax.fori_loop(0, K1 // IDXG, body_at(NS * K0 + s * K1), 0)

        plsc.subcore_barrier()
        pltpu.sync_copy(acc_sh.at[pl.ds(s * RPT, RPT)],
                        out_hbm.at[pl.ds(c * NPAD + s * RPT, RPT)])

    return scatter_kernel


def _dinv_block(degp_ref):
    d = degp_ref[0, :, :1] + degp_ref[1, :, :1] + 1.0
    return lax.rsqrt(d)


def _first_body(x_ref, w_ref, degp_ref, o_ref):
    dinv = _dinv_block(degp_ref)
    o_ref[...] = jnp.dot(x_ref[...], w_ref[...],
                         preferred_element_type=jnp.float32) * dinv


def _mid_body(p_ref, hs_ref, b_ref, w_ref, degp_ref, o_ref):
    dinv = _dinv_block(degp_ref)
    h = dinv * (p_ref[0] + p_ref[1] + hs_ref[...]) + b_ref[...]
    h = jnp.maximum(h, 0.0)
    o_ref[...] = jnp.dot(h, w_ref[...],
                         preferred_element_type=jnp.float32) * dinv


def _final_body(p_ref, hs_ref, b_ref, degp_ref, o_ref):
    dinv = _dinv_block(degp_ref)
    o_ref[...] = dinv * (p_ref[0] + p_ref[1] + hs_ref[...]) + b_ref[...]


def _row_spec(d):
    return pl.BlockSpec((BLK, d), lambda i: (i, 0))


def _part_spec(d):
    return pl.BlockSpec((2, BLK, d), lambda i: (0, i, 0))


def _full_spec(d0, d1):
    return pl.BlockSpec((d0, d1), lambda i: (0, 0))


def _tc_first(x, w, degp):
    return pl.pallas_call(
        _first_body,
        grid=(NPAD // BLK,),
        in_specs=[_row_spec(NFEAT), _full_spec(NFEAT, NHID), _part_spec(NHID)],
        out_specs=_row_spec(NHID),
        out_shape=jax.ShapeDtypeStruct((NPAD, NHID), jnp.float32),
    )(x, w, degp)


def _tc_mid(p, hs, b, w, degp, dout):
    din = hs.shape[1]
    return pl.pallas_call(
        _mid_body,
        grid=(NPAD // BLK,),
        in_specs=[_part_spec(din), _row_spec(din), _full_spec(1, din),
                  _full_spec(din, dout), _part_spec(NHID)],
        out_specs=_row_spec(dout),
        out_shape=jax.ShapeDtypeStruct((NPAD, dout), jnp.float32),
    )(p, hs, b, w, degp)


def _tc_final(p, hs, b, degp):
    d = hs.shape[1]
    return pl.pallas_call(
        _final_body,
        grid=(NPAD // BLK,),
        in_specs=[_part_spec(d), _row_spec(d), _full_spec(1, d),
                  _part_spec(NHID)],
        out_specs=_row_spec(d),
        out_shape=jax.ShapeDtypeStruct((NPAD, d), jnp.float32),
    )(p, hs, b, degp)


def kernel(x, edge_index, W0, b0, W1, b1, W2, b2):
    row = edge_index[0]
    col = edge_index[1]
    pad = EPAD - E
    rowc = jnp.concatenate(
        [row, jnp.full((pad,), DUMMY, jnp.int32)]).reshape(NW * KCH, CH)
    colc = jnp.concatenate(
        [col, jnp.zeros((pad,), jnp.int32)]).reshape(NW * KCH, CH)

    zeros128 = jnp.zeros((NPAD, NHID), jnp.float32)
    ones128 = jnp.ones((CH, NHID), jnp.float32)

    degp = _degree_kernel()(rowc, zeros128, ones128).reshape(2, NPAD, NHID)

    xp = jnp.pad(x, ((0, NPAD - N), (0, 0)))
    W2p = jnp.pad(W2, ((0, 0), (0, 128 - NCLASS)))
    b2p = jnp.pad(b2, (0, 128 - NCLASS))

    scat128 = _scatter_kernel(NHID)

    hs0 = _tc_first(xp, W0, degp)
    p1 = scat128(hs0, colc, rowc, zeros128).reshape(2, NPAD, NHID)
    hs1 = _tc_mid(p1, hs0, b0.reshape(1, -1), W1, degp, NHID)
    p2 = scat128(hs1, colc, rowc, zeros128).reshape(2, NPAD, NHID)
    hs2 = _tc_mid(p2, hs1, b1.reshape(1, -1), W2p, degp, 128)
    p3 = scat128(hs2, colc, rowc, zeros128).reshape(2, NPAD, 128)
    out = _tc_final(p3, hs2, b2p.reshape(1, -1), degp)
    return out[:N, :NCLASS]
